# baseline (speedup 1.0000x reference)
import functools

import jax
import jax.numpy as jnp
from jax import lax
from jax.experimental import pallas as pl
from jax.experimental.pallas import tpu as pltpu
from jax.experimental.pallas import tpu_sc as plsc

N_NODES = 10000
N_EDGES = 320000
NFEAT = 128
NHID = 128
NCLASS = 64

NC = 2
NS = 16
NW = NC * NS
B = 128
EPT = 10240
NB = EPT // B
E_PAD = NW * EPT
NBUF = 2
NPAD = 10240
ROW_BLK = 128

_mesh = plsc.VectorSubcoreMesh(
    core_axis_name="c", subcore_axis_name="s", num_cores=NC, num_subcores=NS)


@functools.partial(
    pl.kernel,
    out_type=jax.ShapeDtypeStruct((NC, NPAD), jnp.float32),
    mesh=_mesh,
    scratch_types=[
        pltpu.VMEM((NB, B), jnp.int32),
        pltpu.VMEM((B,), jnp.float32),
        pltpu.VMEM_SHARED((NPAD,), jnp.float32),
    ],
)
def _deg_kernel(dst_hbm, zeros_hbm, deg_out, idx_v, ones_v, deg_sh):
    cid = lax.axis_index("c")
    sid = lax.axis_index("s")
    wid = sid * NC + cid
    for i in range(B // 16):
        ones_v[pl.ds(16 * i, 16)] = jnp.ones((16,), jnp.float32)

    @pl.when(sid == 0)
    def _():
        pltpu.sync_copy(zeros_hbm, deg_sh)

    plsc.subcore_barrier()
    pltpu.sync_copy(dst_hbm.at[wid], idx_v)

    def body(b):
        pltpu.sync_copy(ones_v, deg_sh.at[idx_v.at[b]], add=True)

    pl.loop(0, NB)(body)
    plsc.subcore_barrier()

    @pl.when(sid == 0)
    def _():
        pltpu.sync_copy(deg_sh, deg_out.at[cid])


@functools.partial(
    pl.kernel,
    out_type=jax.ShapeDtypeStruct((NC, NPAD, NHID), jnp.float32),
    mesh=_mesh,
    scratch_types=[
        pltpu.VMEM((NB, B), jnp.int32),
        [pltpu.VMEM((B,), jnp.int32) for _ in range(NBUF)],
        [pltpu.VMEM((B, NHID), jnp.float32) for _ in range(NBUF)],
        [pltpu.SemaphoreType.DMA for _ in range(NBUF)],
        [pltpu.SemaphoreType.DMA for _ in range(NBUF)],
        pltpu.VMEM_SHARED((NPAD, NHID), jnp.float32),
    ],
)
def _edge_kernel(src_hbm, dst_hbm, g_hbm, zeros_hbm, acc_out,
                 src_v, dstb, rows, gsems, dsems, acc_sh):
    cid = lax.axis_index("c")
    sid = lax.axis_index("s")
    wid = sid * NC + cid

    @pl.when(sid == 0)
    def _():
        pltpu.sync_copy(zeros_hbm, acc_sh)

    plsc.subcore_barrier()
    pltpu.sync_copy(src_hbm.at[wid], src_v)

    def issue(i, j):
        pltpu.async_copy(dst_hbm.at[wid * NB + i], dstb[j], dsems[j])
        pltpu.async_copy(g_hbm.at[src_v.at[i]], rows[j], gsems[j])

    for j in range(NBUF):
        issue(j, j)

    def body(b):
        for j in range(NBUF):
            i = b + j
            pltpu.make_async_copy(dst_hbm.at[wid * NB + i], dstb[j], dsems[j]).wait()
            pltpu.make_async_copy(g_hbm.at[src_v.at[i]], rows[j], gsems[j]).wait()
            pltpu.sync_copy(rows[j], acc_sh.at[dstb[j]], add=True)
            nxt = i + NBUF

            @pl.when(nxt < NB)
            def _():
                issue(nxt, j)

    pl.loop(0, NB, step=NBUF)(body)
    plsc.subcore_barrier()

    @pl.when(sid == 0)
    def _():
        pltpu.sync_copy(acc_sh, acc_out.at[cid])


def _mm_body(x_ref, deg_ref, w1_ref, g_ref):
    deg = deg_ref[0, :] + deg_ref[1, :] + 1.0
    dinv = lax.rsqrt(jnp.maximum(deg, 1.0))
    h = jnp.dot(x_ref[...], w1_ref[...], preferred_element_type=jnp.float32)
    g_ref[...] = h * dinv[:, None]


def _mm_call(x_pad, deg, W1):
    grid = NPAD // ROW_BLK
    return pl.pallas_call(
        _mm_body,
        grid=(grid,),
        in_specs=[
            pl.BlockSpec((ROW_BLK, NFEAT), lambda i: (i, 0)),
            pl.BlockSpec((NC, ROW_BLK), lambda i: (0, i)),
            pl.BlockSpec((NFEAT, NHID), lambda i: (0, 0)),
        ],
        out_specs=pl.BlockSpec((ROW_BLK, NHID), lambda i: (i, 0)),
        out_shape=jax.ShapeDtypeStruct((NPAD, NHID), jnp.float32),
    )(x_pad, deg, W1)


def _ep_body(acc_ref, g_ref, deg_ref, b1_ref, w2_ref, b2_ref, out_ref):
    deg = deg_ref[0, :] + deg_ref[1, :] + 1.0
    dinv = lax.rsqrt(jnp.maximum(deg, 1.0))
    z = dinv[:, None] * (acc_ref[0] + acc_ref[1] + g_ref[...]) + b1_ref[0, :][None, :]
    a = jnp.maximum(z, 0.0)
    logits = jnp.dot(a, w2_ref[...], preferred_element_type=jnp.float32)
    logits = logits + b2_ref[0, :][None, :]
    m = jnp.max(logits, axis=1, keepdims=True)
    lse = jnp.log(jnp.sum(jnp.exp(logits - m), axis=1, keepdims=True)) + m
    out_ref[...] = logits - lse


def _ep_call(acc, g, deg, b1, W2, b2):
    grid = NPAD // ROW_BLK
    return pl.pallas_call(
        _ep_body,
        grid=(grid,),
        in_specs=[
            pl.BlockSpec((NC, ROW_BLK, NHID), lambda i: (0, i, 0)),
            pl.BlockSpec((ROW_BLK, NHID), lambda i: (i, 0)),
            pl.BlockSpec((NC, ROW_BLK), lambda i: (0, i)),
            pl.BlockSpec((1, NHID), lambda i: (0, 0)),
            pl.BlockSpec((NHID, NCLASS), lambda i: (0, 0)),
            pl.BlockSpec((1, NCLASS), lambda i: (0, 0)),
        ],
        out_specs=pl.BlockSpec((ROW_BLK, NCLASS), lambda i: (i, 0)),
        out_shape=jax.ShapeDtypeStruct((NPAD, NCLASS), jnp.float32),
    )(acc, g, deg, b1, W2, b2)


def kernel(x, adj, W1, b1, W2, b2):
    src = adj[0].astype(jnp.int32)
    dst = adj[1].astype(jnp.int32)
    pad = jnp.full((E_PAD - N_EDGES,), N_NODES, dtype=jnp.int32)
    src_t = jnp.concatenate([src, pad]).reshape(NW, NB, B)
    dst_t = jnp.concatenate([dst, pad]).reshape(NW, NB, B)
    dst_t2 = dst_t.reshape(NW * NB, B)

    zeros_n = jnp.zeros((NPAD,), jnp.float32)
    zeros_nf = jnp.zeros((NPAD, NHID), jnp.float32)
    x_pad = jnp.zeros((NPAD, NFEAT), jnp.float32).at[:N_NODES].set(x)

    deg = _deg_kernel(dst_t, zeros_n)
    g = _mm_call(x_pad, deg, W1)
    acc = _edge_kernel(src_t, dst_t2, g, zeros_nf)
    out = _ep_call(acc, g, deg, b1.reshape(1, NHID), W2, b2.reshape(1, NCLASS))
    return out[:N_NODES]


# speedup vs baseline: 1.2374x; 1.2374x over previous
import functools

import jax
import jax.numpy as jnp
from jax import lax
from jax.experimental import pallas as pl
from jax.experimental.pallas import tpu as pltpu
from jax.experimental.pallas import tpu_sc as plsc

N_NODES = 10000
N_EDGES = 320000
NFEAT = 128
NHID = 128
NCLASS = 64
FH = NHID // 2

NC = 2
NS = 16
NW = NC * NS
B = 128
NBUF = 8
EPT = 20480
NB = EPT // B
E_PAD = NS * EPT
DEPT = E_PAD // NW
DNB = DEPT // B
NPAD = 10240
ROW_BLK = 128

_mesh = plsc.VectorSubcoreMesh(
    core_axis_name="c", subcore_axis_name="s", num_cores=NC, num_subcores=NS)


@functools.partial(
    pl.kernel,
    out_type=jax.ShapeDtypeStruct((NC, NPAD), jnp.float32),
    mesh=_mesh,
    scratch_types=[
        pltpu.VMEM((DNB, B), jnp.int32),
        pltpu.VMEM((B,), jnp.float32),
        pltpu.VMEM_SHARED((NPAD,), jnp.float32),
    ],
)
def _deg_kernel(dst_hbm, zeros_hbm, deg_out, idx_v, ones_v, deg_sh):
    cid = lax.axis_index("c")
    sid = lax.axis_index("s")
    wid = sid * NC + cid
    for i in range(B // 16):
        ones_v[pl.ds(16 * i, 16)] = jnp.ones((16,), jnp.float32)

    @pl.when(sid == 0)
    def _():
        pltpu.sync_copy(zeros_hbm, deg_sh)

    plsc.subcore_barrier()
    pltpu.sync_copy(dst_hbm.at[wid], idx_v)

    def body(b):
        pltpu.sync_copy(ones_v, deg_sh.at[idx_v.at[b]], add=True)

    pl.loop(0, DNB)(body)
    plsc.subcore_barrier()

    @pl.when(sid == 0)
    def _():
        pltpu.sync_copy(deg_sh, deg_out.at[cid])


@functools.partial(
    pl.kernel,
    out_type=jax.ShapeDtypeStruct((NC, NPAD, FH), jnp.float32),
    mesh=_mesh,
    scratch_types=[
        pltpu.VMEM((NB, B), jnp.int32),
        [pltpu.VMEM((B,), jnp.int32) for _ in range(NBUF)],
        [pltpu.VMEM((B, FH), jnp.float32) for _ in range(NBUF)],
        [pltpu.SemaphoreType.DMA for _ in range(NBUF)],
        [pltpu.SemaphoreType.DMA for _ in range(NBUF)],
        pltpu.VMEM_SHARED((NPAD, FH), jnp.float32),
    ],
    compiler_params=pltpu.CompilerParams(use_tc_tiling_on_sc=False),
)
def _edge_kernel(src_hbm, dst_hbm, g2_hbm, zeros_hbm, acc_out,
                 src_v, dstb, rows, gsems, dsems, acc_sh):
    cid = lax.axis_index("c")
    sid = lax.axis_index("s")

    @pl.when(sid == 0)
    def _():
        pltpu.sync_copy(zeros_hbm, acc_sh)

    plsc.subcore_barrier()
    pltpu.sync_copy(src_hbm.at[sid], src_v)
    g_half = g2_hbm.at[cid]

    def issue(i, j):
        pltpu.async_copy(dst_hbm.at[sid * NB + i], dstb[j], dsems[j])
        pltpu.async_copy(g_half.at[src_v.at[i]], rows[j], gsems[j])

    for j in range(NBUF):
        issue(j, j)

    def body(b):
        for j in range(NBUF):
            i = b + j
            pltpu.make_async_copy(dst_hbm.at[sid * NB + i], dstb[j], dsems[j]).wait()
            pltpu.make_async_copy(g_half.at[src_v.at[i]], rows[j], gsems[j]).wait()
            pltpu.sync_copy(rows[j], acc_sh.at[dstb[j]], add=True)
            nxt = i + NBUF

            @pl.when(nxt < NB)
            def _():
                issue(nxt, j)

    pl.loop(0, NB, step=NBUF)(body)
    plsc.subcore_barrier()

    @pl.when(sid == 0)
    def _():
        pltpu.sync_copy(acc_sh, acc_out.at[cid])


def _mm_body(x_ref, deg_ref, w1_ref, g2_ref):
    deg = deg_ref[0, :] + deg_ref[1, :] + 1.0
    dinv = lax.rsqrt(jnp.maximum(deg, 1.0))
    h = jnp.dot(x_ref[...], w1_ref[...], preferred_element_type=jnp.float32)
    g = h * dinv[:, None]
    g2_ref[0] = g[:, :FH]
    g2_ref[1] = g[:, FH:]


def _mm_call(x_pad, deg, W1):
    grid = NPAD // ROW_BLK
    return pl.pallas_call(
        _mm_body,
        grid=(grid,),
        in_specs=[
            pl.BlockSpec((ROW_BLK, NFEAT), lambda i: (i, 0)),
            pl.BlockSpec((NC, ROW_BLK), lambda i: (0, i)),
            pl.BlockSpec((NFEAT, NHID), lambda i: (0, 0)),
        ],
        out_specs=pl.BlockSpec((NC, ROW_BLK, FH), lambda i: (0, i, 0)),
        out_shape=jax.ShapeDtypeStruct((NC, NPAD, FH), jnp.float32),
    )(x_pad, deg, W1)


def _ep_body(acc_ref, g2_ref, deg_ref, b1_ref, w2_ref, b2_ref, out_ref):
    deg = deg_ref[0, :] + deg_ref[1, :] + 1.0
    dinv = lax.rsqrt(jnp.maximum(deg, 1.0))
    zl = dinv[:, None] * (acc_ref[0] + g2_ref[0]) + b1_ref[0, :FH][None, :]
    zr = dinv[:, None] * (acc_ref[1] + g2_ref[1]) + b1_ref[0, FH:][None, :]
    al = jnp.maximum(zl, 0.0)
    ar = jnp.maximum(zr, 0.0)
    logits = (jnp.dot(al, w2_ref[:FH, :], preferred_element_type=jnp.float32)
              + jnp.dot(ar, w2_ref[FH:, :], preferred_element_type=jnp.float32)
              + b2_ref[0, :][None, :])
    m = jnp.max(logits, axis=1, keepdims=True)
    lse = jnp.log(jnp.sum(jnp.exp(logits - m), axis=1, keepdims=True)) + m
    out_ref[...] = logits - lse


def _ep_call(acc, g2, deg, b1, W2, b2):
    grid = NPAD // ROW_BLK
    return pl.pallas_call(
        _ep_body,
        grid=(grid,),
        in_specs=[
            pl.BlockSpec((NC, ROW_BLK, FH), lambda i: (0, i, 0)),
            pl.BlockSpec((NC, ROW_BLK, FH), lambda i: (0, i, 0)),
            pl.BlockSpec((NC, ROW_BLK), lambda i: (0, i)),
            pl.BlockSpec((1, NHID), lambda i: (0, 0)),
            pl.BlockSpec((NHID, NCLASS), lambda i: (0, 0)),
            pl.BlockSpec((1, NCLASS), lambda i: (0, 0)),
        ],
        out_specs=pl.BlockSpec((ROW_BLK, NCLASS), lambda i: (i, 0)),
        out_shape=jax.ShapeDtypeStruct((NPAD, NCLASS), jnp.float32),
    )(acc, g2, deg, b1, W2, b2)


def kernel(x, adj, W1, b1, W2, b2):
    src = adj[0].astype(jnp.int32)
    dst = adj[1].astype(jnp.int32)
    pad = jnp.full((E_PAD - N_EDGES,), N_NODES, dtype=jnp.int32)
    src_t = jnp.concatenate([src, pad]).reshape(NS, NB, B)
    dst_pad = jnp.concatenate([dst, pad])
    dst_deg = dst_pad.reshape(NW, DNB, B)
    dst_t2 = dst_pad.reshape(NS * NB, B)

    zeros_n = jnp.zeros((NPAD,), jnp.float32)
    zeros_nf = jnp.zeros((NPAD, FH), jnp.float32)
    x_pad = jnp.zeros((NPAD, NFEAT), jnp.float32).at[:N_NODES].set(x)

    deg = _deg_kernel(dst_deg, zeros_n)
    g2 = _mm_call(x_pad, deg, W1)
    acc = _edge_kernel(src_t, dst_t2, g2, zeros_nf)
    out = _ep_call(acc, g2, deg, b1.reshape(1, NHID), W2, b2.reshape(1, NCLASS))
    return out[:N_NODES]


# speedup vs baseline: 1.3976x; 1.1294x over previous
import functools

import jax
import jax.numpy as jnp
from jax import lax
from jax.experimental import pallas as pl
from jax.experimental.pallas import tpu as pltpu
from jax.experimental.pallas import tpu_sc as plsc

N_NODES = 10000
N_EDGES = 320000
NFEAT = 128
NHID = 128
NCLASS = 64
FH = NHID // 2

NC = 2
NS = 16
NW = NC * NS
B = 128
NBUF = 8
EPT = 20480
NB = EPT // B
E_PAD = NS * EPT
DEPT = E_PAD // NW
DNB = DEPT // B
NPAD = 10240
ROW_BLK = 128

_mesh = plsc.VectorSubcoreMesh(
    core_axis_name="c", subcore_axis_name="s", num_cores=NC, num_subcores=NS)


@functools.partial(
    pl.kernel,
    out_type=jax.ShapeDtypeStruct((NC, NPAD), jnp.float32),
    mesh=_mesh,
    scratch_types=[
        pltpu.VMEM((DNB, B), jnp.int32),
        pltpu.VMEM((B,), jnp.float32),
        pltpu.VMEM_SHARED((NPAD,), jnp.float32),
    ],
)
def _deg_kernel(dst_hbm, zeros_hbm, deg_out, idx_v, ones_v, deg_sh):
    cid = lax.axis_index("c")
    sid = lax.axis_index("s")
    wid = sid * NC + cid
    for i in range(B // 16):
        ones_v[pl.ds(16 * i, 16)] = jnp.ones((16,), jnp.float32)

    @pl.when(sid == 0)
    def _():
        pltpu.sync_copy(zeros_hbm, deg_sh)

    plsc.subcore_barrier()
    pltpu.sync_copy(dst_hbm.at[wid], idx_v)

    def body(b):
        pltpu.sync_copy(ones_v, deg_sh.at[idx_v.at[b]], add=True)

    pl.loop(0, DNB)(body)
    plsc.subcore_barrier()

    @pl.when(sid == 0)
    def _():
        pltpu.sync_copy(deg_sh, deg_out.at[cid])


@functools.partial(
    pl.kernel,
    out_type=jax.ShapeDtypeStruct((NC, NPAD, FH), jnp.float32),
    mesh=_mesh,
    scratch_types=[
        pltpu.VMEM((NB, B), jnp.int32),
        [pltpu.VMEM((B,), jnp.int32) for _ in range(NBUF)],
        [pltpu.VMEM((B, FH), jnp.float32) for _ in range(NBUF)],
        [pltpu.SemaphoreType.DMA for _ in range(NBUF)],
        [pltpu.SemaphoreType.DMA for _ in range(NBUF)],
        pltpu.VMEM_SHARED((NPAD, FH), jnp.float32),
    ],
    compiler_params=pltpu.CompilerParams(use_tc_tiling_on_sc=False),
)
def _edge_kernel(src_hbm, dst_hbm, g2_hbm, zeros_hbm, acc_out,
                 src_v, dstb, rows, gsems, dsems, acc_sh):
    cid = lax.axis_index("c")
    sid = lax.axis_index("s")

    @pl.when(sid == 0)
    def _():
        pltpu.sync_copy(zeros_hbm, acc_sh)

    plsc.subcore_barrier()
    pltpu.sync_copy(src_hbm.at[sid], src_v)
    g_half = g2_hbm.at[cid]

    def issue(i, j):
        pltpu.async_copy(dst_hbm.at[sid * NB + i], dstb[j], dsems[j])
        pltpu.async_copy(g_half.at[src_v.at[i]], rows[j], gsems[j])

    for j in range(NBUF):
        issue(j, j)

    def body(b):
        for j in range(NBUF):
            i = b + j
            pltpu.make_async_copy(dst_hbm.at[sid * NB + i], dstb[j], dsems[j]).wait()
            pltpu.make_async_copy(g_half.at[src_v.at[i]], rows[j], gsems[j]).wait()
            pltpu.sync_copy(rows[j], acc_sh.at[dstb[j]], add=True)
            nxt = i + NBUF

            @pl.when(nxt < NB)
            def _():
                issue(nxt, j)

    pl.loop(0, NB, step=NBUF)(body)
    plsc.subcore_barrier()

    @pl.when(sid == 0)
    def _():
        pltpu.sync_copy(acc_sh, acc_out.at[cid])


def _mm_body(x_ref, deg_ref, w1_ref, g2_ref):
    deg = deg_ref[0, :] + deg_ref[1, :] + 1.0
    dinv = lax.rsqrt(jnp.maximum(deg, 1.0))
    h = jnp.dot(x_ref[...], w1_ref[...], preferred_element_type=jnp.float32)
    g = h * dinv[:, None]
    g2_ref[0] = g[:, :FH]
    g2_ref[1] = g[:, FH:]


def _mm_call(x_pad, deg, W1):
    blk = 1024
    grid = NPAD // blk
    return pl.pallas_call(
        _mm_body,
        grid=(grid,),
        in_specs=[
            pl.BlockSpec((blk, NFEAT), lambda i: (i, 0)),
            pl.BlockSpec((NC, blk), lambda i: (0, i)),
            pl.BlockSpec((NFEAT, NHID), lambda i: (0, 0)),
        ],
        out_specs=pl.BlockSpec((NC, blk, FH), lambda i: (0, i, 0)),
        out_shape=jax.ShapeDtypeStruct((NC, NPAD, FH), jnp.float32),
    )(x_pad, deg, W1)


def _ep_body(acc_ref, g2_ref, deg_ref, b1_ref, w2_ref, b2_ref, out_ref):
    deg = deg_ref[0, :] + deg_ref[1, :] + 1.0
    dinv = lax.rsqrt(jnp.maximum(deg, 1.0))
    zl = dinv[:, None] * (acc_ref[0] + g2_ref[0]) + b1_ref[0, :FH][None, :]
    zr = dinv[:, None] * (acc_ref[1] + g2_ref[1]) + b1_ref[0, FH:][None, :]
    al = jnp.maximum(zl, 0.0)
    ar = jnp.maximum(zr, 0.0)
    logits = (jnp.dot(al, w2_ref[:FH, :], preferred_element_type=jnp.float32)
              + jnp.dot(ar, w2_ref[FH:, :], preferred_element_type=jnp.float32)
              + b2_ref[0, :][None, :])
    m = jnp.max(logits, axis=1, keepdims=True)
    lse = jnp.log(jnp.sum(jnp.exp(logits - m), axis=1, keepdims=True)) + m
    out_ref[...] = logits - lse


def _ep_call(acc, g2, deg, b1, W2, b2):
    blk = 1024
    grid = NPAD // blk
    return pl.pallas_call(
        _ep_body,
        grid=(grid,),
        in_specs=[
            pl.BlockSpec((NC, blk, FH), lambda i: (0, i, 0)),
            pl.BlockSpec((NC, blk, FH), lambda i: (0, i, 0)),
            pl.BlockSpec((NC, blk), lambda i: (0, i)),
            pl.BlockSpec((1, NHID), lambda i: (0, 0)),
            pl.BlockSpec((NHID, NCLASS), lambda i: (0, 0)),
            pl.BlockSpec((1, NCLASS), lambda i: (0, 0)),
        ],
        out_specs=pl.BlockSpec((blk, NCLASS), lambda i: (i, 0)),
        out_shape=jax.ShapeDtypeStruct((N_NODES, NCLASS), jnp.float32),
    )(acc, g2, deg, b1, W2, b2)


def kernel(x, adj, W1, b1, W2, b2):
    src = adj[0].astype(jnp.int32)
    dst = adj[1].astype(jnp.int32)
    pad = jnp.full((E_PAD - N_EDGES,), N_NODES, dtype=jnp.int32)
    src_t = jnp.concatenate([src, pad]).reshape(NS, NB, B)
    dst_pad = jnp.concatenate([dst, pad])
    dst_deg = dst_pad.reshape(NW, DNB, B)
    dst_t2 = dst_pad.reshape(NS * NB, B)

    zeros_n = jnp.zeros((NPAD,), jnp.float32)
    zeros_nf = jnp.zeros((NPAD, FH), jnp.float32)
    x_pad = jnp.zeros((NPAD, NFEAT), jnp.float32).at[:N_NODES].set(x)

    deg = _deg_kernel(dst_deg, zeros_n)
    g2 = _mm_call(x_pad, deg, W1)
    acc = _edge_kernel(src_t, dst_t2, g2, zeros_nf)
    out = _ep_call(acc, g2, deg, b1.reshape(1, NHID), W2, b2.reshape(1, NCLASS))
    return out


# speedup vs baseline: 1.5575x; 1.1144x over previous
import functools

import jax
import jax.numpy as jnp
from jax import lax
from jax.experimental import pallas as pl
from jax.experimental.pallas import tpu as pltpu
from jax.experimental.pallas import tpu_sc as plsc

N_NODES = 10000
N_EDGES = 320000
NFEAT = 128
NHID = 128
NCLASS = 64
FH = NHID // 2

NC = 2
NS = 16
NW = NC * NS
B = 128
NBUF = 8
EPT = 20480
NB = EPT // B
E_PAD = NS * EPT
DEPT = E_PAD // NW
DNB = DEPT // B
NPAD = 10240
ROW_BLK = 128

_mesh = plsc.VectorSubcoreMesh(
    core_axis_name="c", subcore_axis_name="s", num_cores=NC, num_subcores=NS)


@functools.partial(
    pl.kernel,
    out_type=jax.ShapeDtypeStruct((NC, NPAD), jnp.float32),
    mesh=_mesh,
    scratch_types=[
        pltpu.VMEM((DNB, B), jnp.int32),
        pltpu.VMEM((B,), jnp.float32),
        pltpu.VMEM_SHARED((NPAD,), jnp.float32),
    ],
)
def _deg_kernel(dst_hbm, zeros_hbm, deg_out, idx_v, ones_v, deg_sh):
    cid = lax.axis_index("c")
    sid = lax.axis_index("s")
    wid = sid * NC + cid
    for i in range(B // 16):
        ones_v[pl.ds(16 * i, 16)] = jnp.ones((16,), jnp.float32)

    @pl.when(sid == 0)
    def _():
        pltpu.sync_copy(zeros_hbm, deg_sh)

    plsc.subcore_barrier()
    pltpu.sync_copy(dst_hbm.at[wid], idx_v)

    def body(b):
        pltpu.sync_copy(ones_v, deg_sh.at[idx_v.at[b]], add=True)

    pl.loop(0, DNB)(body)
    plsc.subcore_barrier()

    @pl.when(sid == 0)
    def _():
        pltpu.sync_copy(deg_sh, deg_out.at[cid])


@functools.partial(
    pl.kernel,
    out_type=jax.ShapeDtypeStruct((NC, NPAD, FH), jnp.float32),
    mesh=_mesh,
    scratch_types=[
        pltpu.VMEM((NB, B), jnp.int32),
        [pltpu.VMEM((B,), jnp.int32) for _ in range(NBUF)],
        [pltpu.VMEM((B, FH), jnp.float32) for _ in range(NBUF)],
        [pltpu.SemaphoreType.DMA for _ in range(NBUF)],
        [pltpu.SemaphoreType.DMA for _ in range(NBUF)],
        pltpu.VMEM_SHARED((NPAD, FH), jnp.float32),
    ],
    compiler_params=pltpu.CompilerParams(use_tc_tiling_on_sc=False),
)
def _edge_kernel(src_hbm, dst_hbm, g2_hbm, zeros_hbm, acc_out,
                 src_v, dstb, rows, gsems, dsems, acc_sh):
    cid = lax.axis_index("c")
    sid = lax.axis_index("s")

    @pl.when(sid == 0)
    def _():
        pltpu.sync_copy(zeros_hbm, acc_sh)

    plsc.subcore_barrier()
    pltpu.sync_copy(src_hbm.at[sid], src_v)
    g_half = g2_hbm.at[cid]

    def issue(i, j):
        pltpu.async_copy(dst_hbm.at[sid * NB + i], dstb[j], dsems[j])
        pltpu.async_copy(g_half.at[src_v.at[i]], rows[j], gsems[j])

    for j in range(NBUF):
        issue(j, j)

    def body(b):
        for j in range(NBUF):
            i = b + j
            pltpu.make_async_copy(dst_hbm.at[sid * NB + i], dstb[j], dsems[j]).wait()
            pltpu.make_async_copy(g_half.at[src_v.at[i]], rows[j], gsems[j]).wait()
            pltpu.sync_copy(rows[j], acc_sh.at[dstb[j]], add=True)
            nxt = i + NBUF

            @pl.when(nxt < NB)
            def _():
                issue(nxt, j)

    pl.loop(0, NB, step=NBUF)(body)
    plsc.subcore_barrier()

    @pl.when(sid == 0)
    def _():
        pltpu.sync_copy(acc_sh, acc_out.at[cid])


def _dinv_eo(deg_ref):
    de = deg_ref[0, :, 0] + deg_ref[1, :, 0] + 1.0
    do = deg_ref[0, :, 1] + deg_ref[1, :, 1] + 1.0
    return (lax.rsqrt(jnp.maximum(de, 1.0)), lax.rsqrt(jnp.maximum(do, 1.0)))


def _mm_body(x_ref, deg_ref, w1_ref, g2_ref):
    dinv_e, dinv_o = _dinv_eo(deg_ref)
    he = jnp.dot(x_ref[:, 0, :], w1_ref[...], preferred_element_type=jnp.float32)
    ho = jnp.dot(x_ref[:, 1, :], w1_ref[...], preferred_element_type=jnp.float32)
    ge = he * dinv_e[:, None]
    go = ho * dinv_o[:, None]
    g2_ref[0] = jnp.concatenate([ge[:, :FH], go[:, :FH]], axis=1)
    g2_ref[1] = jnp.concatenate([ge[:, FH:], go[:, FH:]], axis=1)


def _mm_call(x_pad, deg, W1):
    blk2 = 512
    grid = (NPAD // 2) // blk2
    x3 = x_pad.reshape(NPAD // 2, 2, NFEAT)
    deg3 = deg.reshape(NC, NPAD // 2, 2)
    return pl.pallas_call(
        _mm_body,
        grid=(grid,),
        in_specs=[
            pl.BlockSpec((blk2, 2, NFEAT), lambda i: (i, 0, 0)),
            pl.BlockSpec((NC, blk2, 2), lambda i: (0, i, 0)),
            pl.BlockSpec((NFEAT, NHID), lambda i: (0, 0)),
        ],
        out_specs=pl.BlockSpec((NC, blk2, NHID), lambda i: (0, i, 0)),
        out_shape=jax.ShapeDtypeStruct((NC, NPAD // 2, NHID), jnp.float32),
    )(x3, deg3, W1)


def _lsm(logits):
    m = jnp.max(logits, axis=1, keepdims=True)
    return logits - (jnp.log(jnp.sum(jnp.exp(logits - m), axis=1, keepdims=True)) + m)


def _ep_body(acc_ref, g2_ref, deg_ref, b1_ref, w2_ref, b2_ref, out_ref):
    dinv_e, dinv_o = _dinv_eo(deg_ref)
    fe = jnp.concatenate([acc_ref[0][:, :FH] + g2_ref[0][:, :FH],
                          acc_ref[1][:, :FH] + g2_ref[1][:, :FH]], axis=1)
    fo = jnp.concatenate([acc_ref[0][:, FH:] + g2_ref[0][:, FH:],
                          acc_ref[1][:, FH:] + g2_ref[1][:, FH:]], axis=1)
    ze = dinv_e[:, None] * fe + b1_ref[0, :][None, :]
    zo = dinv_o[:, None] * fo + b1_ref[0, :][None, :]
    le = (jnp.dot(jnp.maximum(ze, 0.0), w2_ref[...],
                  preferred_element_type=jnp.float32) + b2_ref[0, :][None, :])
    lo = (jnp.dot(jnp.maximum(zo, 0.0), w2_ref[...],
                  preferred_element_type=jnp.float32) + b2_ref[0, :][None, :])
    out_ref[...] = jnp.concatenate([_lsm(le), _lsm(lo)], axis=1)


def _ep_call(acc2, g2t, deg, b1, W2, b2):
    blk2 = 512
    grid = (NPAD // 2) // blk2
    deg3 = deg.reshape(NC, NPAD // 2, 2)
    return pl.pallas_call(
        _ep_body,
        grid=(grid,),
        in_specs=[
            pl.BlockSpec((NC, blk2, NHID), lambda i: (0, i, 0)),
            pl.BlockSpec((NC, blk2, NHID), lambda i: (0, i, 0)),
            pl.BlockSpec((NC, blk2, 2), lambda i: (0, i, 0)),
            pl.BlockSpec((1, NHID), lambda i: (0, 0)),
            pl.BlockSpec((NHID, NCLASS), lambda i: (0, 0)),
            pl.BlockSpec((1, NCLASS), lambda i: (0, 0)),
        ],
        out_specs=pl.BlockSpec((blk2, 2 * NCLASS), lambda i: (i, 0)),
        out_shape=jax.ShapeDtypeStruct((N_NODES // 2, 2 * NCLASS), jnp.float32),
    )(acc2, g2t, deg3, b1, W2, b2)


def kernel(x, adj, W1, b1, W2, b2):
    src = adj[0].astype(jnp.int32)
    dst = adj[1].astype(jnp.int32)
    pad = jnp.full((E_PAD - N_EDGES,), N_NODES, dtype=jnp.int32)
    src_t = jnp.concatenate([src, pad]).reshape(NS, NB, B)
    dst_pad = jnp.concatenate([dst, pad])
    dst_deg = dst_pad.reshape(NW, DNB, B)
    dst_t2 = dst_pad.reshape(NS * NB, B)

    zeros_n = jnp.zeros((NPAD,), jnp.float32)
    zeros_nf = jnp.zeros((NPAD, FH), jnp.float32)
    x_pad = jnp.zeros((NPAD, NFEAT), jnp.float32).at[:N_NODES].set(x)

    deg = _deg_kernel(dst_deg, zeros_n)
    g2t = _mm_call(x_pad, deg, W1)
    g2 = g2t.reshape(NC, NPAD, FH)
    acc = _edge_kernel(src_t, dst_t2, g2, zeros_nf)
    acc2 = acc.reshape(NC, NPAD // 2, NHID)
    out = _ep_call(acc2, g2t, deg, b1.reshape(1, NHID), W2, b2.reshape(1, NCLASS))
    return out.reshape(N_NODES, NCLASS)


# speedup vs baseline: 1.5576x; 1.0001x over previous
import functools

import jax
import jax.numpy as jnp
from jax import lax
from jax.experimental import pallas as pl
from jax.experimental.pallas import tpu as pltpu
from jax.experimental.pallas import tpu_sc as plsc

N_NODES = 10000
N_EDGES = 320000
NFEAT = 128
NHID = 128
NCLASS = 64
FH = NHID // 2

NC = 2
NS = 16
NW = NC * NS
B = 128
NBUF = 8
EPT = 20480
NB = EPT // B
E_PAD = NS * EPT
DEPT = E_PAD // NW
DNB = DEPT // B
NPAD = 10240
ROW_BLK = 128

_mesh = plsc.VectorSubcoreMesh(
    core_axis_name="c", subcore_axis_name="s", num_cores=NC, num_subcores=NS)


@functools.partial(
    pl.kernel,
    out_type=jax.ShapeDtypeStruct((NC, NPAD), jnp.float32),
    mesh=_mesh,
    scratch_types=[
        pltpu.VMEM((DNB, B), jnp.int32),
        pltpu.VMEM((B,), jnp.float32),
        [pltpu.SemaphoreType.DMA for _ in range(4)],
        pltpu.VMEM_SHARED((NPAD,), jnp.float32),
    ],
)
def _deg_kernel(dst_hbm, zeros_hbm, deg_out, idx_v, ones_v, dsems, deg_sh):
    cid = lax.axis_index("c")
    sid = lax.axis_index("s")
    wid = sid * NC + cid
    for i in range(B // 16):
        ones_v[pl.ds(16 * i, 16)] = jnp.ones((16,), jnp.float32)

    @pl.when(sid == 0)
    def _():
        pltpu.sync_copy(zeros_hbm, deg_sh)

    plsc.subcore_barrier()
    pltpu.sync_copy(dst_hbm.at[wid], idx_v)

    for j in range(4):
        pltpu.async_copy(ones_v, deg_sh.at[idx_v.at[j]], dsems[j], add=True)

    def body(b):
        for j in range(4):
            i = b + j
            pltpu.make_async_copy(ones_v, deg_sh.at[idx_v.at[i]], dsems[j]).wait()
            nxt = i + 4

            @pl.when(nxt < DNB)
            def _():
                pltpu.async_copy(ones_v, deg_sh.at[idx_v.at[nxt]], dsems[j], add=True)

    pl.loop(0, DNB, step=4)(body)
    plsc.subcore_barrier()

    @pl.when(sid == 0)
    def _():
        pltpu.sync_copy(deg_sh, deg_out.at[cid])


@functools.partial(
    pl.kernel,
    out_type=jax.ShapeDtypeStruct((NC, NPAD, FH), jnp.float32),
    mesh=_mesh,
    scratch_types=[
        pltpu.VMEM((NB, B), jnp.int32),
        [pltpu.VMEM((B,), jnp.int32) for _ in range(NBUF)],
        [pltpu.VMEM((B, FH), jnp.float32) for _ in range(NBUF)],
        [pltpu.SemaphoreType.DMA for _ in range(NBUF)],
        [pltpu.SemaphoreType.DMA for _ in range(NBUF)],
        pltpu.VMEM_SHARED((NPAD, FH), jnp.float32),
    ],
    compiler_params=pltpu.CompilerParams(use_tc_tiling_on_sc=False),
)
def _edge_kernel(src_hbm, dst_hbm, g2_hbm, zeros_hbm, acc_out,
                 src_v, dstb, rows, gsems, dsems, acc_sh):
    cid = lax.axis_index("c")
    sid = lax.axis_index("s")

    @pl.when(sid == 0)
    def _():
        pltpu.sync_copy(zeros_hbm, acc_sh)

    plsc.subcore_barrier()
    pltpu.sync_copy(src_hbm.at[sid], src_v)
    g_half = g2_hbm.at[cid]

    def issue(i, j):
        pltpu.async_copy(dst_hbm.at[sid * NB + i], dstb[j], dsems[j])
        pltpu.async_copy(g_half.at[src_v.at[i]], rows[j], gsems[j])

    for j in range(NBUF):
        issue(j, j)

    def body(b):
        for j in range(NBUF):
            i = b + j
            pltpu.make_async_copy(dst_hbm.at[sid * NB + i], dstb[j], dsems[j]).wait()
            pltpu.make_async_copy(g_half.at[src_v.at[i]], rows[j], gsems[j]).wait()
            pltpu.sync_copy(rows[j], acc_sh.at[dstb[j]], add=True)
            nxt = i + NBUF

            @pl.when(nxt < NB)
            def _():
                issue(nxt, j)

    pl.loop(0, NB, step=NBUF)(body)
    plsc.subcore_barrier()

    @pl.when(sid == 0)
    def _():
        pltpu.sync_copy(acc_sh, acc_out.at[cid])


def _dinv_eo(deg_ref):
    de = deg_ref[0, :, 0] + deg_ref[1, :, 0] + 1.0
    do = deg_ref[0, :, 1] + deg_ref[1, :, 1] + 1.0
    return (lax.rsqrt(jnp.maximum(de, 1.0)), lax.rsqrt(jnp.maximum(do, 1.0)))


def _mm_body(x_ref, deg_ref, w1_ref, g2_ref):
    dinv_e, dinv_o = _dinv_eo(deg_ref)
    he = jnp.dot(x_ref[:, 0, :], w1_ref[...], preferred_element_type=jnp.float32)
    ho = jnp.dot(x_ref[:, 1, :], w1_ref[...], preferred_element_type=jnp.float32)
    ge = he * dinv_e[:, None]
    go = ho * dinv_o[:, None]
    g2_ref[0] = jnp.concatenate([ge[:, :FH], go[:, :FH]], axis=1)
    g2_ref[1] = jnp.concatenate([ge[:, FH:], go[:, FH:]], axis=1)


def _mm_call(x_pad, deg, W1):
    blk2 = 512
    grid = (NPAD // 2) // blk2
    x3 = x_pad.reshape(NPAD // 2, 2, NFEAT)
    deg3 = deg.reshape(NC, NPAD // 2, 2)
    return pl.pallas_call(
        _mm_body,
        grid=(grid,),
        in_specs=[
            pl.BlockSpec((blk2, 2, NFEAT), lambda i: (i, 0, 0)),
            pl.BlockSpec((NC, blk2, 2), lambda i: (0, i, 0)),
            pl.BlockSpec((NFEAT, NHID), lambda i: (0, 0)),
        ],
        out_specs=pl.BlockSpec((NC, blk2, NHID), lambda i: (0, i, 0)),
        out_shape=jax.ShapeDtypeStruct((NC, NPAD // 2, NHID), jnp.float32),
    )(x3, deg3, W1)


def _lsm(logits):
    m = jnp.max(logits, axis=1, keepdims=True)
    return logits - (jnp.log(jnp.sum(jnp.exp(logits - m), axis=1, keepdims=True)) + m)


def _ep_body(acc_ref, g2_ref, deg_ref, b1_ref, w2_ref, b2_ref, out_ref):
    dinv_e, dinv_o = _dinv_eo(deg_ref)
    fe = jnp.concatenate([acc_ref[0][:, :FH] + g2_ref[0][:, :FH],
                          acc_ref[1][:, :FH] + g2_ref[1][:, :FH]], axis=1)
    fo = jnp.concatenate([acc_ref[0][:, FH:] + g2_ref[0][:, FH:],
                          acc_ref[1][:, FH:] + g2_ref[1][:, FH:]], axis=1)
    ze = dinv_e[:, None] * fe + b1_ref[0, :][None, :]
    zo = dinv_o[:, None] * fo + b1_ref[0, :][None, :]
    le = (jnp.dot(jnp.maximum(ze, 0.0), w2_ref[...],
                  preferred_element_type=jnp.float32) + b2_ref[0, :][None, :])
    lo = (jnp.dot(jnp.maximum(zo, 0.0), w2_ref[...],
                  preferred_element_type=jnp.float32) + b2_ref[0, :][None, :])
    out_ref[...] = jnp.concatenate([_lsm(le), _lsm(lo)], axis=1)


def _ep_call(acc2, g2t, deg, b1, W2, b2):
    blk2 = 512
    grid = (NPAD // 2) // blk2
    deg3 = deg.reshape(NC, NPAD // 2, 2)
    return pl.pallas_call(
        _ep_body,
        grid=(grid,),
        in_specs=[
            pl.BlockSpec((NC, blk2, NHID), lambda i: (0, i, 0)),
            pl.BlockSpec((NC, blk2, NHID), lambda i: (0, i, 0)),
            pl.BlockSpec((NC, blk2, 2), lambda i: (0, i, 0)),
            pl.BlockSpec((1, NHID), lambda i: (0, 0)),
            pl.BlockSpec((NHID, NCLASS), lambda i: (0, 0)),
            pl.BlockSpec((1, NCLASS), lambda i: (0, 0)),
        ],
        out_specs=pl.BlockSpec((blk2, 2 * NCLASS), lambda i: (i, 0)),
        out_shape=jax.ShapeDtypeStruct((N_NODES // 2, 2 * NCLASS), jnp.float32),
    )(acc2, g2t, deg3, b1, W2, b2)


def kernel(x, adj, W1, b1, W2, b2):
    src = adj[0].astype(jnp.int32)
    dst = adj[1].astype(jnp.int32)
    pad = jnp.full((E_PAD - N_EDGES,), N_NODES, dtype=jnp.int32)
    src_t = jnp.concatenate([src, pad]).reshape(NS, NB, B)
    dst_pad = jnp.concatenate([dst, pad])
    dst_deg = dst_pad.reshape(NW, DNB, B)
    dst_t2 = dst_pad.reshape(NS * NB, B)

    zeros_n = jnp.zeros((NPAD,), jnp.float32)
    zeros_nf = jnp.zeros((NPAD, FH), jnp.float32)
    x_pad = jnp.zeros((NPAD, NFEAT), jnp.float32).at[:N_NODES].set(x)

    deg = _deg_kernel(dst_deg, zeros_n)
    g2t = _mm_call(x_pad, deg, W1)
    g2 = g2t.reshape(NC, NPAD, FH)
    acc = _edge_kernel(src_t, dst_t2, g2, zeros_nf)
    acc2 = acc.reshape(NC, NPAD // 2, NHID)
    out = _ep_call(acc2, g2t, deg, b1.reshape(1, NHID), W2, b2.reshape(1, NCLASS))
    return out.reshape(N_NODES, NCLASS)
